# Initial kernel scaffold; baseline (speedup 1.0000x reference)
import jax
import jax.numpy as jnp
from jax.experimental import pallas as pl


def kernel(x, table):
    raise NotImplementedError("write your pallas kernel here")


# speedup vs baseline: 4.1923x; 4.1923x over previous
import functools

import jax
import jax.numpy as jnp
from jax import lax
from jax.experimental import pallas as pl
from jax.experimental.pallas import tpu as pltpu
from jax.experimental.pallas import tpu_sc as plsc

EMBED_DIM = 32
NUM_CORES = 2
NUM_SUBCORES = 16
NUM_WORKERS = NUM_CORES * NUM_SUBCORES
B_TOTAL = 4096 * 200
ROWS_PER_WORKER = B_TOTAL // NUM_WORKERS
CHUNK = 128
CHUNKS_PER_WORKER = ROWS_PER_WORKER // CHUNK


def _emb_body(table_hbm, idx_hbm, out_hbm, idx_v, rows_v, gsem):
    wid = lax.axis_index("s") * NUM_CORES + lax.axis_index("c")
    base = wid * ROWS_PER_WORKER
    pltpu.sync_copy(idx_hbm.at[wid], idx_v)

    def step(j, carry):
        pltpu.async_copy(table_hbm.at[idx_v.at[j]], rows_v, gsem).wait()
        pltpu.sync_copy(rows_v, out_hbm.at[pl.ds(base + j * CHUNK, CHUNK)])
        return carry

    lax.fori_loop(0, CHUNKS_PER_WORKER, step, 0)


_emb = pl.kernel(
    _emb_body,
    out_type=jax.ShapeDtypeStruct((B_TOTAL, EMBED_DIM), jnp.float32),
    mesh=plsc.VectorSubcoreMesh(core_axis_name="c", subcore_axis_name="s"),
    scratch_types=[
        pltpu.VMEM((CHUNKS_PER_WORKER, CHUNK), jnp.int32),
        pltpu.VMEM((CHUNK, EMBED_DIM), jnp.float32),
        pltpu.SemaphoreType.DMA,
    ],
    compiler_params=pltpu.CompilerParams(use_tc_tiling_on_sc=False),
)


@jax.jit
def kernel(x, table):
    idx = x.reshape(NUM_WORKERS, CHUNKS_PER_WORKER, CHUNK).astype(jnp.int32)
    out = _emb(table, idx)
    return out.reshape(x.shape[0], x.shape[1], EMBED_DIM)


# speedup vs baseline: 5.3104x; 1.2667x over previous
import jax
import jax.numpy as jnp
from jax import lax
from jax.experimental import pallas as pl
from jax.experimental.pallas import tpu as pltpu
from jax.experimental.pallas import tpu_sc as plsc

EMBED_DIM = 32
NUM_CORES = 2
NUM_SUBCORES = 16
NUM_WORKERS = NUM_CORES * NUM_SUBCORES
B_TOTAL = 4096 * 200
ROWS_PER_WORKER = B_TOTAL // NUM_WORKERS
CHUNK = 128
CHUNKS_PER_WORKER = ROWS_PER_WORKER // CHUNK
KG = 10
GROUP_ROWS = KG * CHUNK
NGROUPS = CHUNKS_PER_WORKER // KG


def _emb_body(table_hbm, idx_hbm, out_hbm,
              idx_v, rows0, rows1, gsem0, gsem1, osem0, osem1):
    wid = lax.axis_index("s") * NUM_CORES + lax.axis_index("c")
    base = wid * ROWS_PER_WORKER
    rows = (rows0, rows1)
    gsem = (gsem0, gsem1)
    osem = (osem0, osem1)

    pltpu.sync_copy(idx_hbm.at[wid], idx_v)

    def fire(g, b):
        for k in range(KG):
            pltpu.async_copy(
                table_hbm.at[idx_v.at[g * KG + k]],
                rows[b].at[pl.ds(k * CHUNK, CHUNK)],
                gsem[b],
            )

    def drain_gathers(b):
        pltpu.make_async_copy(
            out_hbm.at[pl.ds(base, GROUP_ROWS)], rows[b], gsem[b]
        ).wait()

    def out_start(g, b):
        pltpu.async_copy(
            rows[b], out_hbm.at[pl.ds(base + g * GROUP_ROWS, GROUP_ROWS)],
            osem[b],
        )

    def out_wait(b):
        pltpu.make_async_copy(
            rows[b], out_hbm.at[pl.ds(base, GROUP_ROWS)], osem[b]
        ).wait()

    fire(0, 0)

    def phase(g, b, first, last):
        if not first:
            out_wait(1 - b)
        if not last:
            fire(g + 1, 1 - b)
        drain_gathers(b)
        out_start(g, b)

    phase(0, 0, first=True, last=False)
    phase(1, 1, first=False, last=False)

    def steady(i, carry):
        g = 2 * i
        phase(g, 0, first=False, last=False)
        phase(g + 1, 1, first=False, last=False)
        return carry

    lax.fori_loop(1, NGROUPS // 2 - 1, steady, 0)

    phase(NGROUPS - 2, 0, first=False, last=False)
    phase(NGROUPS - 1, 1, first=False, last=True)
    out_wait(1)


_emb = pl.kernel(
    _emb_body,
    out_type=jax.ShapeDtypeStruct((B_TOTAL, EMBED_DIM), jnp.float32),
    mesh=plsc.VectorSubcoreMesh(core_axis_name="c", subcore_axis_name="s"),
    scratch_types=[
        pltpu.VMEM((CHUNKS_PER_WORKER, CHUNK), jnp.int32),
        pltpu.VMEM((GROUP_ROWS, EMBED_DIM), jnp.float32),
        pltpu.VMEM((GROUP_ROWS, EMBED_DIM), jnp.float32),
        pltpu.SemaphoreType.DMA,
        pltpu.SemaphoreType.DMA,
        pltpu.SemaphoreType.DMA,
        pltpu.SemaphoreType.DMA,
    ],
    compiler_params=pltpu.CompilerParams(use_tc_tiling_on_sc=False),
)


@jax.jit
def kernel(x, table):
    idx = x.reshape(NUM_WORKERS, CHUNKS_PER_WORKER, CHUNK).astype(jnp.int32)
    out = _emb(table, idx)
    return out.reshape(x.shape[0], x.shape[1], EMBED_DIM)
